# baseline (speedup 1.0000x reference)
import math

import jax
import jax.numpy as jnp
from jax import lax
from jax.experimental import pallas as pl
from jax.experimental.pallas import tpu as pltpu

_SMOOTHING = 0.1
_CONFIDENCE = 1.0 - _SMOOTHING
_PAD = 0
_R = 32
_NBUF = 4


def _body(batch, v, eps, c1):
    nblk = batch // _R

    def body(t_ref, x_hbm, out_ref, buf, sems):
        def start(b):
            slot = lax.rem(b, _NBUF)
            pltpu.make_async_copy(
                x_hbm.at[pl.ds(b * _R, _R), :], buf.at[slot], sems.at[slot]
            ).start()

        def wait(b):
            slot = lax.rem(b, _NBUF)
            pltpu.make_async_copy(
                x_hbm.at[pl.ds(b * _R, _R), :], buf.at[slot], sems.at[slot]
            ).wait()

        for k in range(_NBUF - 1):
            start(jnp.int32(k))

        cols = lax.broadcasted_iota(jnp.int32, (_R, v), 1)

        def step(b, acc):
            wait(b)
            xv = buf[lax.rem(b, _NBUF)]
            t = t_ref[pl.ds(b * _R, _R), :]

            @pl.when(b + _NBUF - 1 < nblk)
            def _():
                start(b + _NBUF - 1)

            s = jnp.sum(xv, axis=1, keepdims=True)
            g = jnp.sum(jnp.where(cols == t, xv, 0.0), axis=1, keepdims=True)
            x0 = xv[:, 0:1]
            valid = (t != _PAD).astype(jnp.float32)
            per_row = valid * (c1 - eps * s + eps * x0
                               + (eps - _CONFIDENCE) * g)
            return acc + jnp.sum(per_row)

        total = lax.fori_loop(0, nblk, step, jnp.float32(0.0))
        out_ref[:, :] = jnp.full((1, 1), 0.0) + total

    return body


def kernel(x, target):
    batch, v = x.shape
    eps = _SMOOTHING / (v - 2)
    c1 = eps * math.log(eps) * (v - 2) + _CONFIDENCE * math.log(_CONFIDENCE)

    t2 = target.astype(jnp.int32).reshape(batch, 1)

    out = pl.pallas_call(
        _body(batch, v, eps, c1),
        in_specs=[
            pl.BlockSpec((batch, 1), lambda: (0, 0)),
            pl.BlockSpec(memory_space=pl.ANY),
        ],
        out_specs=pl.BlockSpec((1, 1), lambda: (0, 0)),
        out_shape=jax.ShapeDtypeStruct((1, 1), jnp.float32),
        scratch_shapes=[
            pltpu.VMEM((_NBUF, _R, v), jnp.float32),
            pltpu.SemaphoreType.DMA((_NBUF,)),
        ],
    )(t2, x)
    return out[0, 0]
